# baseline (speedup 1.0000x reference)
import functools

import jax
import jax.numpy as jnp
from jax import lax
from jax.experimental import pallas as pl
from jax.experimental.pallas import tpu as pltpu
from jax.experimental.pallas import tpu_sc as plsc

N = 10000
E = 320000
D = 128
NC = 2
NS = 16
NW = NC * NS
EPT = E // NW
CH = 128
NF = EPT // CH
TE = EPT - NF * CH
SLAB = 632
LAST = N - (NS - 1) * SLAB
NB = 3
N_DEG = 10240

_MESH = plsc.VectorSubcoreMesh(core_axis_name="c", subcore_axis_name="s")


@functools.partial(
    pl.kernel,
    mesh=_MESH,
    out_type=jax.ShapeDtypeStruct((NW, N_DEG), jnp.float32),
    scratch_types=[
        pltpu.VMEM((EPT,), jnp.int32),
        pltpu.VMEM((N_DEG,), jnp.float32),
    ],
    compiler_params=pltpu.CompilerParams(needs_layout_passes=False),
)
def _deg_sc(col_hbm, out_hbm, col_v, deg_v):
    cid = lax.axis_index("c")
    sid = lax.axis_index("s")
    w = cid * NS + sid
    pltpu.sync_copy(col_hbm.at[pl.ds(w * EPT, EPT)], col_v)

    zero16 = jnp.zeros((16,), jnp.float32)

    def zbody(i, _):
        deg_v[pl.ds(i * 16, 16)] = zero16
        return 0

    lax.fori_loop(0, N_DEG // 16, zbody, 0)

    ones16 = jnp.ones((16,), jnp.float32)

    def body(j, _):
        idx = col_v[pl.ds(j * 16, 16)]
        plsc.addupdate_scatter(deg_v, [idx], ones16)
        return 0

    lax.fori_loop(0, EPT // 16, body, 0)
    pltpu.sync_copy(deg_v, out_hbm.at[w])


@functools.partial(
    pl.kernel,
    mesh=_MESH,
    out_type=jax.ShapeDtypeStruct((NC, N, D), jnp.float32),
    scratch_types=[
        [pltpu.VMEM((CH,), jnp.int32) for _ in range(NB)],
        [pltpu.VMEM((CH,), jnp.int32) for _ in range(NB)],
        [pltpu.VMEM((CH, D), jnp.float32) for _ in range(NB)],
        pltpu.VMEM((TE,), jnp.int32),
        pltpu.VMEM((TE,), jnp.int32),
        pltpu.VMEM_SHARED((N, D), jnp.float32),
        [pltpu.SemaphoreType.DMA for _ in range(NB)],
        [pltpu.SemaphoreType.DMA for _ in range(NB)],
    ],
    compiler_params=pltpu.CompilerParams(needs_layout_passes=False),
)
def _agg_sc(h_hbm, row_hbm, col_hbm, zer_hbm, out_hbm,
            rchs, cchs, gbufs, rch_t, cch_t, shared, isems, gsems):
    cid = lax.axis_index("c")
    sid = lax.axis_index("s")
    base = (cid * NS + sid) * EPT

    @pl.when(sid < NS - 1)
    def _():
        pltpu.sync_copy(zer_hbm, shared.at[pl.ds(sid * SLAB, SLAB)])

    @pl.when(sid == NS - 1)
    def _():
        pltpu.sync_copy(zer_hbm.at[pl.ds(0, LAST)],
                        shared.at[pl.ds((NS - 1) * SLAB, LAST)])

    plsc.subcore_barrier()

    def issue_idx(j, k):
        pltpu.async_copy(row_hbm.at[pl.ds(base + j * CH, CH)], rchs[k], isems[k])
        pltpu.async_copy(col_hbm.at[pl.ds(base + j * CH, CH)], cchs[k], isems[k])

    def wait_idx(k):
        pltpu.make_async_copy(row_hbm.at[pl.ds(base, CH)], rchs[k],
                              isems[k]).wait()
        pltpu.make_async_copy(col_hbm.at[pl.ds(base, CH)], cchs[k],
                              isems[k]).wait()

    def issue_gather(k):
        pltpu.async_copy(h_hbm.at[rchs[k]], gbufs[k], gsems[k])

    def wait_gather(k):
        pltpu.make_async_copy(h_hbm.at[rchs[k]], gbufs[k], gsems[k]).wait()

    def scatter(k):
        pltpu.sync_copy(gbufs[k], shared.at[cchs[k]], add=True)

    for j in range(NB):
        issue_idx(j, j)
    wait_idx(0)
    issue_gather(0)
    wait_idx(1)
    issue_gather(1)

    def body(g, _):
        j0 = g * NB
        for k in range(NB):
            wait_gather(k)
            scatter(k)
            issue_idx(j0 + k + NB, k)
            wait_idx((k + NB - 1) % NB)
            issue_gather((k + NB - 1) % NB)
        return 0

    lax.fori_loop(0, (NF - NB) // NB, body, 0)
    wait_gather(0)
    scatter(0)
    wait_idx(2)
    issue_gather(2)
    wait_gather(1)
    scatter(1)
    wait_gather(2)
    scatter(2)

    pltpu.sync_copy(row_hbm.at[pl.ds(base + NF * CH, TE)], rch_t)
    pltpu.sync_copy(col_hbm.at[pl.ds(base + NF * CH, TE)], cch_t)
    pltpu.async_copy(h_hbm.at[rch_t], gbufs[0].at[pl.ds(0, TE)],
                     gsems[0]).wait()
    pltpu.sync_copy(gbufs[0].at[pl.ds(0, TE)], shared.at[cch_t], add=True)

    plsc.subcore_barrier()

    @pl.when(sid < NS - 1)
    def _():
        pltpu.sync_copy(shared.at[pl.ds(sid * SLAB, SLAB)],
                        out_hbm.at[cid, pl.ds(sid * SLAB, SLAB)])

    @pl.when(sid == NS - 1)
    def _():
        pltpu.sync_copy(shared.at[pl.ds((NS - 1) * SLAB, LAST)],
                        out_hbm.at[cid, pl.ds((NS - 1) * SLAB, LAST)])


_GRID = 25
_BM = N // _GRID


def _dense1(x_ref, w1_ref, degt_ref, h1s_ref, dis1_ref, dis2_ref):
    h1 = jnp.dot(x_ref[...], w1_ref[...], preferred_element_type=jnp.float32)
    indeg = jnp.sum(degt_ref[...], axis=1, keepdims=True)
    dis1 = lax.rsqrt(indeg + 3.0)
    dis2 = lax.rsqrt(indeg + 2.0)
    h1s_ref[...] = h1 * dis1
    dis1_ref[...] = dis1
    dis2_ref[...] = dis2


def _dense2(agg_ref, h1s_ref, dis1_ref, dis2_ref, b1_ref, g_ref, bb_ref,
            w2_ref, h2s_ref):
    a = agg_ref[0] + agg_ref[1]
    d1 = dis1_ref[...]
    out1 = d1 * a + 3.0 * d1 * h1s_ref[...] + b1_ref[...]
    mu = jnp.mean(out1, axis=1, keepdims=True)
    cz = out1 - mu
    var = jnp.mean(cz * cz, axis=1, keepdims=True)
    z = cz * lax.rsqrt(var + 1e-5) * g_ref[...] + bb_ref[...]
    z = jnp.maximum(z, 0.0)
    h2 = jnp.dot(z, w2_ref[...], preferred_element_type=jnp.float32)
    h2s_ref[...] = dis2_ref[...] * h2


def _dense3(agg_ref, h2s_ref, dis2_ref, b2_ref, x_ref, o_ref):
    a = agg_ref[0] + agg_ref[1]
    d2 = dis2_ref[...]
    o_ref[...] = d2 * a + 2.0 * d2 * h2s_ref[...] + b2_ref[...] + x_ref[...]


def _row_spec(minor):
    return pl.BlockSpec((_BM, minor), lambda i: (i, 0))


def _full_spec(shape):
    nd = len(shape)
    return pl.BlockSpec(shape, lambda i: (0,) * nd)


_dense1_call = pl.pallas_call(
    _dense1,
    grid=(_GRID,),
    in_specs=[_row_spec(D), _full_spec((D, D)), _row_spec(32)],
    out_specs=[_row_spec(D), _row_spec(1), _row_spec(1)],
    out_shape=[
        jax.ShapeDtypeStruct((N, D), jnp.float32),
        jax.ShapeDtypeStruct((N, 1), jnp.float32),
        jax.ShapeDtypeStruct((N, 1), jnp.float32),
    ],
)

_agg_spec = pl.BlockSpec((NC, _BM, D), lambda i: (0, i, 0))

_dense2_call = pl.pallas_call(
    _dense2,
    grid=(_GRID,),
    in_specs=[_agg_spec, _row_spec(D), _row_spec(1), _row_spec(1),
              _full_spec((1, D)), _full_spec((1, D)), _full_spec((1, D)),
              _full_spec((D, D))],
    out_specs=_row_spec(D),
    out_shape=jax.ShapeDtypeStruct((N, D), jnp.float32),
)

_dense3_call = pl.pallas_call(
    _dense3,
    grid=(_GRID,),
    in_specs=[_agg_spec, _row_spec(D), _row_spec(1), _full_spec((1, D)),
              _row_spec(D)],
    out_specs=_row_spec(D),
    out_shape=jax.ShapeDtypeStruct((N, D), jnp.float32),
)


def kernel(x, edge_index, W1, b1, ln_g, ln_b, W2, b2):
    row = edge_index[0]
    col = edge_index[1]

    degp = _deg_sc(col)
    degt = degp.T[:N]

    h1s, dis1, dis2 = _dense1_call(x, W1, degt)

    zer = jnp.zeros((SLAB, D), jnp.float32)
    agg1 = _agg_sc(h1s, row, col, zer)
    h2s = _dense2_call(agg1, h1s, dis1, dis2,
                       b1.reshape(1, D), ln_g.reshape(1, D),
                       ln_b.reshape(1, D), W2)
    agg2 = _agg_sc(h2s, row, col, zer)
    out = _dense3_call(agg2, h2s, dis2, b2.reshape(1, D), x)
    return out


# speedup vs baseline: 1.0433x; 1.0433x over previous
import functools

import jax
import jax.numpy as jnp
from jax import lax
from jax.experimental import pallas as pl
from jax.experimental.pallas import tpu as pltpu
from jax.experimental.pallas import tpu_sc as plsc

N = 10000
E = 320000
D = 128
NC = 2
NS = 16
NW = NC * NS
EPT = E // NW
CH = 80
NCH = EPT // CH
SLAB = 632
LAST = N - (NS - 1) * SLAB
NB = 3
N_DEG = 10240

_MESH = plsc.VectorSubcoreMesh(core_axis_name="c", subcore_axis_name="s")


@functools.partial(
    pl.kernel,
    mesh=_MESH,
    out_type=jax.ShapeDtypeStruct((NW, N_DEG), jnp.float32),
    scratch_types=[
        pltpu.VMEM((EPT,), jnp.int32),
        pltpu.VMEM((N_DEG,), jnp.float32),
    ],
    compiler_params=pltpu.CompilerParams(needs_layout_passes=False),
)
def _deg_sc(col_hbm, out_hbm, col_v, deg_v):
    cid = lax.axis_index("c")
    sid = lax.axis_index("s")
    w = cid * NS + sid
    pltpu.sync_copy(col_hbm.at[pl.ds(w * EPT, EPT)], col_v)

    zero16 = jnp.zeros((16,), jnp.float32)

    def zbody(i, _):
        deg_v[pl.ds(i * 16, 16)] = zero16
        return 0

    lax.fori_loop(0, N_DEG // 16, zbody, 0)

    ones16 = jnp.ones((16,), jnp.float32)

    def body(j, _):
        idx = col_v[pl.ds(j * 16, 16)]
        plsc.addupdate_scatter(deg_v, [idx], ones16)
        return 0

    lax.fori_loop(0, EPT // 16, body, 0)
    pltpu.sync_copy(deg_v, out_hbm.at[w])


@functools.partial(
    pl.kernel,
    mesh=_MESH,
    out_type=jax.ShapeDtypeStruct((NC, N, D), jnp.float32),
    scratch_types=[
        pltpu.VMEM((EPT,), jnp.int32),
        [pltpu.VMEM((CH,), jnp.int32) for _ in range(NB)],
        [pltpu.VMEM((CH, D), jnp.float32) for _ in range(NB)],
        pltpu.VMEM_SHARED((N, D), jnp.float32),
        [pltpu.SemaphoreType.DMA for _ in range(NB)],
        [pltpu.SemaphoreType.DMA for _ in range(NB)],
        [pltpu.SemaphoreType.DMA for _ in range(NB)],
    ],
    compiler_params=pltpu.CompilerParams(needs_layout_passes=False),
)
def _agg_sc(h_hbm, row_hbm, col_hbm, zer_hbm, out_hbm,
            row_v, cchs, gbufs, shared, gsems, ssems, isems):
    cid = lax.axis_index("c")
    sid = lax.axis_index("s")
    w = cid * NS + sid
    base = w * EPT
    pltpu.sync_copy(row_hbm.at[pl.ds(base, EPT)], row_v)

    @pl.when(sid < NS - 1)
    def _():
        pltpu.sync_copy(zer_hbm, shared.at[pl.ds(sid * SLAB, SLAB)])

    @pl.when(sid == NS - 1)
    def _():
        pltpu.sync_copy(zer_hbm.at[pl.ds(0, LAST)],
                        shared.at[pl.ds((NS - 1) * SLAB, LAST)])

    plsc.subcore_barrier()

    def issue_gather(j, k):
        pltpu.async_copy(h_hbm.at[row_v.at[pl.ds(j * CH, CH)]],
                         gbufs[k], gsems[k])

    def wait_gather(k):
        pltpu.make_async_copy(h_hbm.at[row_v.at[pl.ds(0, CH)]],
                              gbufs[k], gsems[k]).wait()

    def issue_cidx(j, k):
        pltpu.async_copy(col_hbm.at[pl.ds(base + j * CH, CH)],
                         cchs[k], isems[k])

    def wait_cidx(k):
        pltpu.make_async_copy(col_hbm.at[pl.ds(base, CH)],
                              cchs[k], isems[k]).wait()

    def issue_scatter(k):
        pltpu.async_copy(gbufs[k], shared.at[cchs[k]], ssems[k], add=True)

    def wait_scatter(k):
        pltpu.make_async_copy(gbufs[k], shared.at[cchs[k]], ssems[k]).wait()

    issue_gather(0, 0)
    issue_cidx(0, 0)
    issue_gather(1, 1)
    issue_cidx(1, 1)
    wait_gather(0)
    wait_cidx(0)
    issue_scatter(0)
    issue_gather(2, 2)
    issue_cidx(2, 2)

    def body(g, _):
        j0 = g * NB
        for (k, off) in ((1, 1), (2, 2), (0, 3)):
            j = j0 + off
            wait_gather(k)
            wait_cidx(k)
            issue_scatter(k)
            kprev = (k + NB - 1) % NB
            wait_scatter(kprev)
            issue_gather(j + 2, kprev)
            issue_cidx(j + 2, kprev)
        return 0

    lax.fori_loop(0, (NCH - 5) // NB, body, 0)
    wait_gather(1)
    wait_cidx(1)
    issue_scatter(1)
    wait_scatter(0)
    issue_gather(123, 0)
    issue_cidx(123, 0)
    wait_gather(2)
    wait_cidx(2)
    issue_scatter(2)
    wait_scatter(1)
    issue_gather(124, 1)
    issue_cidx(124, 1)
    wait_gather(0)
    wait_cidx(0)
    issue_scatter(0)
    wait_scatter(2)
    wait_gather(1)
    wait_cidx(1)
    issue_scatter(1)
    wait_scatter(0)
    wait_scatter(1)

    plsc.subcore_barrier()

    @pl.when(sid < NS - 1)
    def _():
        pltpu.sync_copy(shared.at[pl.ds(sid * SLAB, SLAB)],
                        out_hbm.at[cid, pl.ds(sid * SLAB, SLAB)])

    @pl.when(sid == NS - 1)
    def _():
        pltpu.sync_copy(shared.at[pl.ds((NS - 1) * SLAB, LAST)],
                        out_hbm.at[cid, pl.ds((NS - 1) * SLAB, LAST)])


_GRID = 25
_BM = N // _GRID


def _dense1(x_ref, w1_ref, degt_ref, h1s_ref, dis1_ref, dis2_ref):
    h1 = jnp.dot(x_ref[...], w1_ref[...], preferred_element_type=jnp.float32)
    indeg = jnp.sum(degt_ref[...], axis=1, keepdims=True)
    dis1 = lax.rsqrt(indeg + 3.0)
    dis2 = lax.rsqrt(indeg + 2.0)
    h1s_ref[...] = h1 * dis1
    dis1_ref[...] = dis1
    dis2_ref[...] = dis2


def _dense2(agg_ref, h1s_ref, dis1_ref, dis2_ref, b1_ref, g_ref, bb_ref,
            w2_ref, h2s_ref):
    a = agg_ref[0] + agg_ref[1]
    d1 = dis1_ref[...]
    out1 = d1 * a + 3.0 * d1 * h1s_ref[...] + b1_ref[...]
    mu = jnp.mean(out1, axis=1, keepdims=True)
    cz = out1 - mu
    var = jnp.mean(cz * cz, axis=1, keepdims=True)
    z = cz * lax.rsqrt(var + 1e-5) * g_ref[...] + bb_ref[...]
    z = jnp.maximum(z, 0.0)
    h2 = jnp.dot(z, w2_ref[...], preferred_element_type=jnp.float32)
    h2s_ref[...] = dis2_ref[...] * h2


def _dense3(agg_ref, h2s_ref, dis2_ref, b2_ref, x_ref, o_ref):
    a = agg_ref[0] + agg_ref[1]
    d2 = dis2_ref[...]
    o_ref[...] = d2 * a + 2.0 * d2 * h2s_ref[...] + b2_ref[...] + x_ref[...]


def _row_spec(minor):
    return pl.BlockSpec((_BM, minor), lambda i: (i, 0))


def _full_spec(shape):
    nd = len(shape)
    return pl.BlockSpec(shape, lambda i: (0,) * nd)


_dense1_call = pl.pallas_call(
    _dense1,
    grid=(_GRID,),
    in_specs=[_row_spec(D), _full_spec((D, D)), _row_spec(32)],
    out_specs=[_row_spec(D), _row_spec(1), _row_spec(1)],
    out_shape=[
        jax.ShapeDtypeStruct((N, D), jnp.float32),
        jax.ShapeDtypeStruct((N, 1), jnp.float32),
        jax.ShapeDtypeStruct((N, 1), jnp.float32),
    ],
)

_agg_spec = pl.BlockSpec((NC, _BM, D), lambda i: (0, i, 0))

_dense2_call = pl.pallas_call(
    _dense2,
    grid=(_GRID,),
    in_specs=[_agg_spec, _row_spec(D), _row_spec(1), _row_spec(1),
              _full_spec((1, D)), _full_spec((1, D)), _full_spec((1, D)),
              _full_spec((D, D))],
    out_specs=_row_spec(D),
    out_shape=jax.ShapeDtypeStruct((N, D), jnp.float32),
)

_dense3_call = pl.pallas_call(
    _dense3,
    grid=(_GRID,),
    in_specs=[_agg_spec, _row_spec(D), _row_spec(1), _full_spec((1, D)),
              _row_spec(D)],
    out_specs=_row_spec(D),
    out_shape=jax.ShapeDtypeStruct((N, D), jnp.float32),
)


def kernel(x, edge_index, W1, b1, ln_g, ln_b, W2, b2):
    row = edge_index[0]
    col = edge_index[1]

    degp = _deg_sc(col)
    degt = degp.T[:N]

    h1s, dis1, dis2 = _dense1_call(x, W1, degt)

    zer = jnp.zeros((SLAB, D), jnp.float32)
    agg1 = _agg_sc(h1s, row, col, zer)
    h2s = _dense2_call(agg1, h1s, dis1, dis2,
                       b1.reshape(1, D), ln_g.reshape(1, D),
                       ln_b.reshape(1, D), W2)
    agg2 = _agg_sc(h2s, row, col, zer)
    out = _dense3_call(agg2, h2s, dis2, b2.reshape(1, D), x)
    return out


# speedup vs baseline: 1.1489x; 1.1012x over previous
import functools

import jax
import jax.numpy as jnp
from jax import lax
from jax.experimental import pallas as pl
from jax.experimental.pallas import tpu as pltpu
from jax.experimental.pallas import tpu_sc as plsc

N = 10000
E = 320000
D = 128
NC = 2
NS = 16
NW = NC * NS
EPT = E // NW
CH = 80
NCH = EPT // CH
SLAB = 632
LAST = N - (NS - 1) * SLAB
NB = 3
N_DEG = 10240

_MESH = plsc.VectorSubcoreMesh(core_axis_name="c", subcore_axis_name="s")


@functools.partial(
    pl.kernel,
    mesh=_MESH,
    out_type=jax.ShapeDtypeStruct((NW, N_DEG), jnp.float32),
    scratch_types=[
        pltpu.VMEM((EPT,), jnp.int32),
        pltpu.VMEM((N_DEG,), jnp.float32),
    ],
    compiler_params=pltpu.CompilerParams(needs_layout_passes=False),
)
def _deg_sc(col_hbm, out_hbm, col_v, deg_v):
    cid = lax.axis_index("c")
    sid = lax.axis_index("s")
    w = cid * NS + sid
    pltpu.sync_copy(col_hbm.at[pl.ds(w * EPT, EPT)], col_v)

    zero16 = jnp.zeros((16,), jnp.float32)

    def zbody(i, _):
        deg_v[pl.ds(i * 16, 16)] = zero16
        return 0

    lax.fori_loop(0, N_DEG // 16, zbody, 0)

    ones16 = jnp.ones((16,), jnp.float32)

    def body(j, _):
        idx = col_v[pl.ds(j * 16, 16)]
        plsc.addupdate_scatter(deg_v, [idx], ones16)
        return 0

    lax.fori_loop(0, EPT // 16, body, 0)
    pltpu.sync_copy(deg_v, out_hbm.at[w])


@functools.partial(
    pl.kernel,
    mesh=_MESH,
    out_type=jax.ShapeDtypeStruct((NC, N, D), jnp.float32),
    scratch_types=[
        pltpu.VMEM((EPT,), jnp.int32),
        [pltpu.VMEM((CH,), jnp.int32) for _ in range(NB)],
        [pltpu.VMEM((CH, D), jnp.float32) for _ in range(NB)],
        pltpu.VMEM_SHARED((N, D), jnp.float32),
        [pltpu.SemaphoreType.DMA for _ in range(NB)],
        [pltpu.SemaphoreType.DMA for _ in range(NB)],
        [pltpu.SemaphoreType.DMA for _ in range(NB)],
    ],
    compiler_params=pltpu.CompilerParams(needs_layout_passes=False),
)
def _agg_sc(h_hbm, row_hbm, col_hbm, zer_hbm, out_hbm,
            row_v, cchs, gbufs, shared, gsems, ssems, isems):
    cid = lax.axis_index("c")
    sid = lax.axis_index("s")
    w = cid * NS + sid
    base = w * EPT
    pltpu.sync_copy(row_hbm.at[pl.ds(base, EPT)], row_v)

    @pl.when(sid < NS - 1)
    def _():
        pltpu.sync_copy(zer_hbm, shared.at[pl.ds(sid * SLAB, SLAB)])

    @pl.when(sid == NS - 1)
    def _():
        pltpu.sync_copy(zer_hbm.at[pl.ds(0, LAST)],
                        shared.at[pl.ds((NS - 1) * SLAB, LAST)])

    plsc.subcore_barrier()

    def issue_gather(j, k):
        pltpu.async_copy(h_hbm.at[row_v.at[pl.ds(j * CH, CH)]],
                         gbufs[k], gsems[k])

    def wait_gather(k):
        pltpu.make_async_copy(h_hbm.at[row_v.at[pl.ds(0, CH)]],
                              gbufs[k], gsems[k]).wait()

    def issue_cidx(j, k):
        pltpu.async_copy(col_hbm.at[pl.ds(base + j * CH, CH)],
                         cchs[k], isems[k])

    def wait_cidx(k):
        pltpu.make_async_copy(col_hbm.at[pl.ds(base, CH)],
                              cchs[k], isems[k]).wait()

    def issue_scatter(k):
        pltpu.async_copy(gbufs[k], shared.at[cchs[k]], ssems[k], add=True)

    def wait_scatter(k):
        pltpu.make_async_copy(gbufs[k], shared.at[cchs[k]], ssems[k]).wait()

    issue_gather(0, 0)
    issue_cidx(0, 0)
    issue_gather(1, 1)
    issue_cidx(1, 1)
    wait_gather(0)
    wait_cidx(0)
    issue_scatter(0)
    issue_gather(2, 2)
    issue_cidx(2, 2)

    def body(g, _):
        j0 = g * NB
        for (k, off) in ((1, 1), (2, 2), (0, 3)):
            j = j0 + off
            wait_gather(k)
            wait_cidx(k)
            issue_scatter(k)
            kprev = (k + NB - 1) % NB
            wait_scatter(kprev)
            issue_gather(j + 2, kprev)
            issue_cidx(j + 2, kprev)
        return 0

    lax.fori_loop(0, (NCH - 5) // NB, body, 0)
    wait_gather(1)
    wait_cidx(1)
    issue_scatter(1)
    wait_scatter(0)
    issue_gather(123, 0)
    issue_cidx(123, 0)
    wait_gather(2)
    wait_cidx(2)
    issue_scatter(2)
    wait_scatter(1)
    issue_gather(124, 1)
    issue_cidx(124, 1)
    wait_gather(0)
    wait_cidx(0)
    issue_scatter(0)
    wait_scatter(2)
    wait_gather(1)
    wait_cidx(1)
    issue_scatter(1)
    wait_scatter(0)
    wait_scatter(1)

    plsc.subcore_barrier()

    @pl.when(sid < NS - 1)
    def _():
        pltpu.sync_copy(shared.at[pl.ds(sid * SLAB, SLAB)],
                        out_hbm.at[cid, pl.ds(sid * SLAB, SLAB)])

    @pl.when(sid == NS - 1)
    def _():
        pltpu.sync_copy(shared.at[pl.ds((NS - 1) * SLAB, LAST)],
                        out_hbm.at[cid, pl.ds((NS - 1) * SLAB, LAST)])


_GRID = 5
_BM = N // _GRID


def _dense1(x_ref, w1_ref, degt_ref, h1s_ref, dis1_ref, dis2_ref):
    h1 = jnp.dot(x_ref[...], w1_ref[...], preferred_element_type=jnp.float32)
    indeg = jnp.sum(degt_ref[...], axis=1, keepdims=True)
    dis1 = lax.rsqrt(indeg + 3.0)
    dis2 = lax.rsqrt(indeg + 2.0)
    h1s_ref[...] = h1 * dis1
    dis1_ref[...] = dis1
    dis2_ref[...] = dis2


def _dense2(agg_ref, h1s_ref, dis1_ref, dis2_ref, b1_ref, g_ref, bb_ref,
            w2_ref, h2s_ref):
    a = agg_ref[0] + agg_ref[1]
    d1 = dis1_ref[...]
    out1 = d1 * a + 3.0 * d1 * h1s_ref[...] + b1_ref[...]
    mu = jnp.mean(out1, axis=1, keepdims=True)
    cz = out1 - mu
    var = jnp.mean(cz * cz, axis=1, keepdims=True)
    z = cz * lax.rsqrt(var + 1e-5) * g_ref[...] + bb_ref[...]
    z = jnp.maximum(z, 0.0)
    h2 = jnp.dot(z, w2_ref[...], preferred_element_type=jnp.float32)
    h2s_ref[...] = dis2_ref[...] * h2


def _dense3(agg_ref, h2s_ref, dis2_ref, b2_ref, x_ref, o_ref):
    a = agg_ref[0] + agg_ref[1]
    d2 = dis2_ref[...]
    o_ref[...] = d2 * a + 2.0 * d2 * h2s_ref[...] + b2_ref[...] + x_ref[...]


def _row_spec(minor):
    return pl.BlockSpec((_BM, minor), lambda i: (i, 0))


def _full_spec(shape):
    nd = len(shape)
    return pl.BlockSpec(shape, lambda i: (0,) * nd)


_dense1_call = pl.pallas_call(
    _dense1,
    grid=(_GRID,),
    in_specs=[_row_spec(D), _full_spec((D, D)), _row_spec(32)],
    out_specs=[_row_spec(D), _row_spec(1), _row_spec(1)],
    out_shape=[
        jax.ShapeDtypeStruct((N, D), jnp.float32),
        jax.ShapeDtypeStruct((N, 1), jnp.float32),
        jax.ShapeDtypeStruct((N, 1), jnp.float32),
    ],
)

_agg_spec = pl.BlockSpec((NC, _BM, D), lambda i: (0, i, 0))

_dense2_call = pl.pallas_call(
    _dense2,
    grid=(_GRID,),
    in_specs=[_agg_spec, _row_spec(D), _row_spec(1), _row_spec(1),
              _full_spec((1, D)), _full_spec((1, D)), _full_spec((1, D)),
              _full_spec((D, D))],
    out_specs=_row_spec(D),
    out_shape=jax.ShapeDtypeStruct((N, D), jnp.float32),
)

_dense3_call = pl.pallas_call(
    _dense3,
    grid=(_GRID,),
    in_specs=[_agg_spec, _row_spec(D), _row_spec(1), _full_spec((1, D)),
              _row_spec(D)],
    out_specs=_row_spec(D),
    out_shape=jax.ShapeDtypeStruct((N, D), jnp.float32),
)


def kernel(x, edge_index, W1, b1, ln_g, ln_b, W2, b2):
    row = edge_index[0]
    col = edge_index[1]

    degp = _deg_sc(col)
    degt = degp.T[:N]

    h1s, dis1, dis2 = _dense1_call(x, W1, degt)

    zer = jnp.zeros((SLAB, D), jnp.float32)
    agg1 = _agg_sc(h1s, row, col, zer)
    h2s = _dense2_call(agg1, h1s, dis1, dis2,
                       b1.reshape(1, D), ln_g.reshape(1, D),
                       ln_b.reshape(1, D), W2)
    agg2 = _agg_sc(h2s, row, col, zer)
    out = _dense3_call(agg2, h2s, dis2, b2.reshape(1, D), x)
    return out


# speedup vs baseline: 1.1568x; 1.0069x over previous
import functools

import jax
import jax.numpy as jnp
from jax import lax
from jax.experimental import pallas as pl
from jax.experimental.pallas import tpu as pltpu
from jax.experimental.pallas import tpu_sc as plsc

N = 10000
E = 320000
D = 128
NC = 2
NS = 16
NW = NC * NS
EPT = E // NW
CH = 80
NCH = EPT // CH
SLAB = 632
LAST = N - (NS - 1) * SLAB
NB = 3
N_DEG = 10240

_MESH = plsc.VectorSubcoreMesh(core_axis_name="c", subcore_axis_name="s")


@functools.partial(
    pl.kernel,
    mesh=_MESH,
    out_type=jax.ShapeDtypeStruct((NW, N_DEG), jnp.float32),
    scratch_types=[
        pltpu.VMEM((EPT,), jnp.int32),
        pltpu.VMEM((N_DEG,), jnp.float32),
    ],
    compiler_params=pltpu.CompilerParams(needs_layout_passes=False),
)
def _deg_sc(col_hbm, out_hbm, col_v, deg_v):
    cid = lax.axis_index("c")
    sid = lax.axis_index("s")
    w = cid * NS + sid
    pltpu.sync_copy(col_hbm.at[pl.ds(w * EPT, EPT)], col_v)

    zero16 = jnp.zeros((16,), jnp.float32)

    def zbody(i, _):
        deg_v[pl.ds(i * 16, 16)] = zero16
        return 0

    lax.fori_loop(0, N_DEG // 16, zbody, 0)

    ones16 = jnp.ones((16,), jnp.float32)

    def body(j, _):
        idx = col_v[pl.ds(j * 16, 16)]
        plsc.addupdate_scatter(deg_v, [idx], ones16)
        return 0

    lax.fori_loop(0, EPT // 16, body, 0)
    pltpu.sync_copy(deg_v, out_hbm.at[w])


@functools.partial(
    pl.kernel,
    mesh=_MESH,
    out_type=jax.ShapeDtypeStruct((NC, N, D), jnp.float32),
    scratch_types=[
        pltpu.VMEM((EPT,), jnp.int32),
        [pltpu.VMEM((CH,), jnp.int32) for _ in range(NB)],
        [pltpu.VMEM((CH, D), jnp.float32) for _ in range(NB)],
        pltpu.VMEM_SHARED((N, D), jnp.float32),
        [pltpu.SemaphoreType.DMA for _ in range(NB)],
        [pltpu.SemaphoreType.DMA for _ in range(NB)],
        [pltpu.SemaphoreType.DMA for _ in range(NB)],
    ],
    compiler_params=pltpu.CompilerParams(needs_layout_passes=False),
)
def _agg_sc(h_hbm, row_hbm, col_hbm, zer_hbm, out_hbm,
            row_v, cchs, gbufs, shared, gsems, ssems, isems):
    cid = lax.axis_index("c")
    sid = lax.axis_index("s")
    w = cid * NS + sid
    base = w * EPT
    pltpu.sync_copy(row_hbm.at[pl.ds(base, EPT)], row_v)

    @pl.when(sid < NS - 1)
    def _():
        pltpu.sync_copy(zer_hbm, shared.at[pl.ds(sid * SLAB, SLAB)])

    @pl.when(sid == NS - 1)
    def _():
        pltpu.sync_copy(zer_hbm.at[pl.ds(0, LAST)],
                        shared.at[pl.ds((NS - 1) * SLAB, LAST)])

    plsc.subcore_barrier()

    def issue_gather(j, k):
        pltpu.async_copy(h_hbm.at[row_v.at[pl.ds(j * CH, CH)]],
                         gbufs[k], gsems[k])

    def wait_gather(k):
        pltpu.make_async_copy(h_hbm.at[row_v.at[pl.ds(0, CH)]],
                              gbufs[k], gsems[k]).wait()

    def issue_cidx(j, k):
        pltpu.async_copy(col_hbm.at[pl.ds(base + j * CH, CH)],
                         cchs[k], isems[k])

    def wait_cidx(k):
        pltpu.make_async_copy(col_hbm.at[pl.ds(base, CH)],
                              cchs[k], isems[k]).wait()

    def issue_scatter(k):
        pltpu.async_copy(gbufs[k], shared.at[cchs[k]], ssems[k], add=True)

    def wait_scatter(k):
        pltpu.make_async_copy(gbufs[k], shared.at[cchs[k]], ssems[k]).wait()

    issue_gather(0, 0)
    issue_cidx(0, 0)
    issue_gather(1, 1)
    issue_cidx(1, 1)
    wait_gather(0)
    wait_cidx(0)
    issue_scatter(0)
    issue_gather(2, 2)
    issue_cidx(2, 2)

    def body(g, _):
        j0 = g * NB
        for (k, off) in ((1, 1), (2, 2), (0, 3)):
            j = j0 + off
            wait_gather(k)
            wait_cidx(k)
            issue_scatter(k)
            kprev = (k + NB - 1) % NB
            wait_scatter(kprev)
            issue_gather(j + 2, kprev)
            issue_cidx(j + 2, kprev)
        return 0

    lax.fori_loop(0, (NCH - 5) // NB, body, 0)
    wait_gather(1)
    wait_cidx(1)
    issue_scatter(1)
    wait_scatter(0)
    issue_gather(123, 0)
    issue_cidx(123, 0)
    wait_gather(2)
    wait_cidx(2)
    issue_scatter(2)
    wait_scatter(1)
    issue_gather(124, 1)
    issue_cidx(124, 1)
    wait_gather(0)
    wait_cidx(0)
    issue_scatter(0)
    wait_scatter(2)
    wait_gather(1)
    wait_cidx(1)
    issue_scatter(1)
    wait_scatter(0)
    wait_scatter(1)

    plsc.subcore_barrier()

    @pl.when(sid < NS - 1)
    def _():
        pltpu.sync_copy(shared.at[pl.ds(sid * SLAB, SLAB)],
                        out_hbm.at[cid, pl.ds(sid * SLAB, SLAB)])

    @pl.when(sid == NS - 1)
    def _():
        pltpu.sync_copy(shared.at[pl.ds((NS - 1) * SLAB, LAST)],
                        out_hbm.at[cid, pl.ds((NS - 1) * SLAB, LAST)])


_GRID = 2
_BM = N // _GRID


def _dense1(x_ref, w1_ref, degt_ref, h1s_ref, dis1_ref, dis2_ref):
    h1 = jnp.dot(x_ref[...], w1_ref[...], preferred_element_type=jnp.float32)
    indeg = jnp.sum(degt_ref[...], axis=1, keepdims=True)
    dis1 = lax.rsqrt(indeg + 3.0)
    dis2 = lax.rsqrt(indeg + 2.0)
    h1s_ref[...] = h1 * dis1
    dis1_ref[...] = dis1
    dis2_ref[...] = dis2


def _dense2(agg_ref, h1s_ref, dis1_ref, dis2_ref, b1_ref, g_ref, bb_ref,
            w2_ref, h2s_ref):
    a = agg_ref[0] + agg_ref[1]
    d1 = dis1_ref[...]
    out1 = d1 * a + 3.0 * d1 * h1s_ref[...] + b1_ref[...]
    mu = jnp.mean(out1, axis=1, keepdims=True)
    cz = out1 - mu
    var = jnp.mean(cz * cz, axis=1, keepdims=True)
    z = cz * lax.rsqrt(var + 1e-5) * g_ref[...] + bb_ref[...]
    z = jnp.maximum(z, 0.0)
    h2 = jnp.dot(z, w2_ref[...], preferred_element_type=jnp.float32)
    h2s_ref[...] = dis2_ref[...] * h2


def _dense3(agg_ref, h2s_ref, dis2_ref, b2_ref, x_ref, o_ref):
    a = agg_ref[0] + agg_ref[1]
    d2 = dis2_ref[...]
    o_ref[...] = d2 * a + 2.0 * d2 * h2s_ref[...] + b2_ref[...] + x_ref[...]


def _row_spec(minor):
    return pl.BlockSpec((_BM, minor), lambda i: (i, 0))


def _full_spec(shape):
    nd = len(shape)
    return pl.BlockSpec(shape, lambda i: (0,) * nd)


_dense1_call = pl.pallas_call(
    _dense1,
    grid=(_GRID,),
    in_specs=[_row_spec(D), _full_spec((D, D)), _row_spec(32)],
    out_specs=[_row_spec(D), _row_spec(1), _row_spec(1)],
    out_shape=[
        jax.ShapeDtypeStruct((N, D), jnp.float32),
        jax.ShapeDtypeStruct((N, 1), jnp.float32),
        jax.ShapeDtypeStruct((N, 1), jnp.float32),
    ],
)

_agg_spec = pl.BlockSpec((NC, _BM, D), lambda i: (0, i, 0))

_dense2_call = pl.pallas_call(
    _dense2,
    grid=(_GRID,),
    in_specs=[_agg_spec, _row_spec(D), _row_spec(1), _row_spec(1),
              _full_spec((1, D)), _full_spec((1, D)), _full_spec((1, D)),
              _full_spec((D, D))],
    out_specs=_row_spec(D),
    out_shape=jax.ShapeDtypeStruct((N, D), jnp.float32),
)

_dense3_call = pl.pallas_call(
    _dense3,
    grid=(_GRID,),
    in_specs=[_agg_spec, _row_spec(D), _row_spec(1), _full_spec((1, D)),
              _row_spec(D)],
    out_specs=_row_spec(D),
    out_shape=jax.ShapeDtypeStruct((N, D), jnp.float32),
)


def kernel(x, edge_index, W1, b1, ln_g, ln_b, W2, b2):
    row = edge_index[0]
    col = edge_index[1]

    degp = _deg_sc(col)
    degt = degp.T[:N]

    h1s, dis1, dis2 = _dense1_call(x, W1, degt)

    zer = jnp.zeros((SLAB, D), jnp.float32)
    agg1 = _agg_sc(h1s, row, col, zer)
    h2s = _dense2_call(agg1, h1s, dis1, dis2,
                       b1.reshape(1, D), ln_g.reshape(1, D),
                       ln_b.reshape(1, D), W2)
    agg2 = _agg_sc(h2s, row, col, zer)
    out = _dense3_call(agg2, h2s, dis2, b2.reshape(1, D), x)
    return out


# speedup vs baseline: 1.1616x; 1.0042x over previous
import functools

import jax
import jax.numpy as jnp
from jax import lax
from jax.experimental import pallas as pl
from jax.experimental.pallas import tpu as pltpu
from jax.experimental.pallas import tpu_sc as plsc

N = 10000
E = 320000
D = 128
NC = 2
NS = 16
NW = NC * NS
EPT = E // NW
CH = 80
NCH = EPT // CH
SLAB = 632
LAST = N - (NS - 1) * SLAB
NB = 3
N_DEG = 10240

_MESH = plsc.VectorSubcoreMesh(core_axis_name="c", subcore_axis_name="s")


@functools.partial(
    pl.kernel,
    mesh=_MESH,
    out_type=jax.ShapeDtypeStruct((NW, N_DEG), jnp.float32),
    scratch_types=[
        pltpu.VMEM((EPT,), jnp.int32),
        pltpu.VMEM((N_DEG,), jnp.float32),
    ],
    compiler_params=pltpu.CompilerParams(needs_layout_passes=False),
)
def _deg_sc(col_hbm, out_hbm, col_v, deg_v):
    cid = lax.axis_index("c")
    sid = lax.axis_index("s")
    w = cid * NS + sid
    pltpu.sync_copy(col_hbm.at[pl.ds(w * EPT, EPT)], col_v)

    zero16 = jnp.zeros((16,), jnp.float32)

    def zbody(i, _):
        deg_v[pl.ds(i * 16, 16)] = zero16
        return 0

    lax.fori_loop(0, N_DEG // 16, zbody, 0)

    ones16 = jnp.ones((16,), jnp.float32)

    def body(j, _):
        idx = col_v[pl.ds(j * 16, 16)]
        plsc.addupdate_scatter(deg_v, [idx], ones16)
        return 0

    lax.fori_loop(0, EPT // 16, body, 0)
    pltpu.sync_copy(deg_v, out_hbm.at[w])


@functools.partial(
    pl.kernel,
    mesh=_MESH,
    out_type=jax.ShapeDtypeStruct((NC, N, D), jnp.float32),
    scratch_types=[
        pltpu.VMEM((EPT,), jnp.int32),
        [pltpu.VMEM((CH,), jnp.int32) for _ in range(NB)],
        [pltpu.VMEM((CH, D), jnp.float32) for _ in range(NB)],
        pltpu.VMEM_SHARED((N, D), jnp.float32),
        [pltpu.SemaphoreType.DMA for _ in range(NB)],
        [pltpu.SemaphoreType.DMA for _ in range(NB)],
        [pltpu.SemaphoreType.DMA for _ in range(NB)],
    ],
    compiler_params=pltpu.CompilerParams(needs_layout_passes=False),
)
def _agg_sc(h_hbm, row_hbm, col_hbm, zer_hbm, out_hbm,
            row_v, cchs, gbufs, shared, gsems, ssems, isems):
    cid = lax.axis_index("c")
    sid = lax.axis_index("s")
    w = cid * NS + sid
    base = w * EPT
    pltpu.sync_copy(row_hbm.at[pl.ds(base, EPT)], row_v)

    @pl.when(sid < NS - 1)
    def _():
        pltpu.sync_copy(zer_hbm, shared.at[pl.ds(sid * SLAB, SLAB)])

    @pl.when(sid == NS - 1)
    def _():
        pltpu.sync_copy(zer_hbm.at[pl.ds(0, LAST)],
                        shared.at[pl.ds((NS - 1) * SLAB, LAST)])

    plsc.subcore_barrier()

    def issue_gather(j, k):
        pltpu.async_copy(h_hbm.at[row_v.at[pl.ds(j * CH, CH)]],
                         gbufs[k], gsems[k])

    def wait_gather(k):
        pltpu.make_async_copy(h_hbm.at[row_v.at[pl.ds(0, CH)]],
                              gbufs[k], gsems[k]).wait()

    def issue_cidx(j, k):
        pltpu.async_copy(col_hbm.at[pl.ds(base + j * CH, CH)],
                         cchs[k], isems[k])

    def wait_cidx(k):
        pltpu.make_async_copy(col_hbm.at[pl.ds(base, CH)],
                              cchs[k], isems[k]).wait()

    def issue_scatter(k):
        pltpu.async_copy(gbufs[k], shared.at[cchs[k]], ssems[k], add=True)

    def wait_scatter(k):
        pltpu.make_async_copy(gbufs[k], shared.at[cchs[k]], ssems[k]).wait()

    issue_gather(0, 0)
    issue_cidx(0, 0)
    issue_gather(1, 1)
    issue_cidx(1, 1)
    wait_gather(0)
    wait_cidx(0)
    issue_scatter(0)
    issue_gather(2, 2)
    issue_cidx(2, 2)

    def body(g, _):
        j0 = g * NB
        for (k, off) in ((1, 1), (2, 2), (0, 3)):
            j = j0 + off
            wait_gather(k)
            wait_cidx(k)
            issue_scatter(k)
            kprev = (k + NB - 1) % NB
            wait_scatter(kprev)
            issue_gather(j + 2, kprev)
            issue_cidx(j + 2, kprev)
        return 0

    lax.fori_loop(0, (NCH - 5) // NB, body, 0)
    wait_gather(1)
    wait_cidx(1)
    issue_scatter(1)
    wait_scatter(0)
    issue_gather(123, 0)
    issue_cidx(123, 0)
    wait_gather(2)
    wait_cidx(2)
    issue_scatter(2)
    wait_scatter(1)
    issue_gather(124, 1)
    issue_cidx(124, 1)
    wait_gather(0)
    wait_cidx(0)
    issue_scatter(0)
    wait_scatter(2)
    wait_gather(1)
    wait_cidx(1)
    issue_scatter(1)
    wait_scatter(0)
    wait_scatter(1)

    plsc.subcore_barrier()

    @pl.when(sid < NS - 1)
    def _():
        pltpu.sync_copy(shared.at[pl.ds(sid * SLAB, SLAB)],
                        out_hbm.at[cid, pl.ds(sid * SLAB, SLAB)])

    @pl.when(sid == NS - 1)
    def _():
        pltpu.sync_copy(shared.at[pl.ds((NS - 1) * SLAB, LAST)],
                        out_hbm.at[cid, pl.ds((NS - 1) * SLAB, LAST)])


_GRID = 2
_BM = N // _GRID


def _dense1(x_ref, w1_ref, degt_ref, h1s_ref, dis1_ref, dis2_ref):
    h1 = jnp.dot(x_ref[...], w1_ref[...], preferred_element_type=jnp.float32)
    indeg = jnp.sum(degt_ref[...], axis=1, keepdims=True)
    dis1 = lax.rsqrt(indeg + 3.0)
    dis2 = lax.rsqrt(indeg + 2.0)
    h1s_ref[...] = h1 * dis1
    dis1_ref[...] = dis1
    dis2_ref[...] = dis2


def _dense2(agg_ref, h1s_ref, dis1_ref, dis2_ref, b1_ref, g_ref, bb_ref,
            w2_ref, h2s_ref):
    a = agg_ref[0] + agg_ref[1]
    d1 = dis1_ref[...]
    out1 = d1 * a + 3.0 * d1 * h1s_ref[...] + b1_ref[...]
    mu = jnp.mean(out1, axis=1, keepdims=True)
    cz = out1 - mu
    var = jnp.mean(cz * cz, axis=1, keepdims=True)
    z = cz * lax.rsqrt(var + 1e-5) * g_ref[...] + bb_ref[...]
    z = jnp.maximum(z, 0.0)
    h2 = jnp.dot(z, w2_ref[...], preferred_element_type=jnp.float32)
    h2s_ref[...] = dis2_ref[...] * h2


def _dense3(agg_ref, h2s_ref, dis2_ref, b2_ref, x_ref, o_ref):
    a = agg_ref[0] + agg_ref[1]
    d2 = dis2_ref[...]
    o_ref[...] = d2 * a + 2.0 * d2 * h2s_ref[...] + b2_ref[...] + x_ref[...]


def _row_spec(minor):
    return pl.BlockSpec((_BM, minor), lambda i: (i, 0))


def _full_spec(shape):
    nd = len(shape)
    return pl.BlockSpec(shape, lambda i: (0,) * nd)


_dense1_call = pl.pallas_call(
    _dense1,
    grid=(_GRID,),
    in_specs=[_row_spec(D), _full_spec((D, D)), _row_spec(32)],
    out_specs=[_row_spec(D), _row_spec(1), _row_spec(1)],
    out_shape=[
        jax.ShapeDtypeStruct((N, D), jnp.float32),
        jax.ShapeDtypeStruct((N, 1), jnp.float32),
        jax.ShapeDtypeStruct((N, 1), jnp.float32),
    ],
)

_agg_spec = pl.BlockSpec((NC, _BM, D), lambda i: (0, i, 0))

_dense2_call = pl.pallas_call(
    _dense2,
    grid=(_GRID,),
    in_specs=[_agg_spec, _row_spec(D), _row_spec(1), _row_spec(1),
              _full_spec((1, D)), _full_spec((1, D)), _full_spec((1, D)),
              _full_spec((D, D))],
    out_specs=_row_spec(D),
    out_shape=jax.ShapeDtypeStruct((N, D), jnp.float32),
)

_dense3_call = pl.pallas_call(
    _dense3,
    grid=(_GRID,),
    in_specs=[_agg_spec, _row_spec(D), _row_spec(1), _full_spec((1, D)),
              _row_spec(D)],
    out_specs=_row_spec(D),
    out_shape=jax.ShapeDtypeStruct((N, D), jnp.float32),
)


def kernel(x, edge_index, W1, b1, ln_g, ln_b, W2, b2):
    row = edge_index[0]
    col = edge_index[1]

    degp = _deg_sc(col)
    degt = degp[:, :N].T

    h1s, dis1, dis2 = _dense1_call(x, W1, degt)

    zer = jnp.zeros((SLAB, D), jnp.float32)
    agg1 = _agg_sc(h1s, row, col, zer)
    h2s = _dense2_call(agg1, h1s, dis1, dis2,
                       b1.reshape(1, D), ln_g.reshape(1, D),
                       ln_b.reshape(1, D), W2)
    agg2 = _agg_sc(h2s, row, col, zer)
    out = _dense3_call(agg2, h2s, dis2, b2.reshape(1, D), x)
    return out
